# Initial kernel scaffold; baseline (speedup 1.0000x reference)
import jax
import jax.numpy as jnp
from jax.experimental import pallas as pl


def kernel(x, edge_index, W1, b1, W2, b2):
    raise NotImplementedError("write your pallas kernel here")


# speedup vs baseline: 84.9540x; 84.9540x over previous
import functools

import jax
import jax.numpy as jnp
from jax import lax
from jax.experimental import pallas as pl
from jax.experimental.pallas import tpu as pltpu
from jax.experimental.pallas import tpu_sc as plsc

N = 10000
NP = 10112
RPT = NP // 16
NR = NP // 8
E = 320000
BLK = 128
NW = 32
NB = 80
E_PAD = NW * NB * BLK
F = 16

_mesh = plsc.VectorSubcoreMesh(core_axis_name="c", subcore_axis_name="s")


def _tc0_body(ei_ref, src_ref, dst_ref):
    fill = jnp.full((E_PAD - E,), N, jnp.int32)
    src_ref[:E] = ei_ref[0]
    src_ref[E:] = fill
    dst_ref[:E] = ei_ref[1]
    dst_ref[E:] = fill


@functools.partial(
    pl.kernel,
    out_type=(
        jax.ShapeDtypeStruct((NP, F), jnp.float32),
        jax.ShapeDtypeStruct((NP, F), jnp.float32),
    ),
    mesh=_mesh,
    compiler_params=pltpu.CompilerParams(use_tc_tiling_on_sc=False),
    scratch_types=[
        pltpu.VMEM((NB, BLK), jnp.int32),
        pltpu.VMEM((BLK,), jnp.float32),
        pltpu.VMEM((640,), jnp.float32),
        pltpu.VMEM((640, F), jnp.float32),
        pltpu.VMEM_SHARED((NP,), jnp.float32),
    ],
)
def _sc_degree(dst_hbm, out0_hbm, out1_hbm, dstv, ones_v, degbuf, repbuf,
               aggsh):
    c = lax.axis_index("c")
    s = lax.axis_index("s")
    wid = c * 16 + s
    pltpu.sync_copy(dst_hbm.at[pl.ds(wid * NB, NB)], dstv)

    def initz(i, carry):
        degbuf[pl.ds(i * 16, 16)] = jnp.zeros((16,), jnp.float32)
        return carry

    lax.fori_loop(0, 40, initz, 0)

    def init1(i, carry):
        ones_v[pl.ds(i * 16, 16)] = jnp.ones((16,), jnp.float32)
        return carry

    lax.fori_loop(0, BLK // 16, init1, 0)
    pltpu.sync_copy(degbuf.at[pl.ds(0, RPT)], aggsh.at[pl.ds(s * RPT, RPT)])
    plsc.subcore_barrier()

    def body(j, carry):
        pltpu.sync_copy(ones_v, aggsh.at[dstv.at[j]], add=True)
        return carry

    lax.fori_loop(0, NB, body, 0)
    plsc.subcore_barrier()

    pltpu.sync_copy(aggsh.at[pl.ds(s * RPT, RPT)], degbuf.at[pl.ds(0, RPT)])

    def repl(g, carry):
        v = degbuf[pl.ds(g * 16, 16)]
        for k in range(16):
            repbuf[g * 16 + k] = jnp.full((F,), v[k], jnp.float32)
        return carry

    lax.fori_loop(0, 40, repl, 0)

    @pl.when(c == 0)
    def _():
        pltpu.sync_copy(repbuf.at[pl.ds(0, RPT)],
                        out0_hbm.at[pl.ds(s * RPT, RPT)])

    @pl.when(c == 1)
    def _():
        pltpu.sync_copy(repbuf.at[pl.ds(0, RPT)],
                        out1_hbm.at[pl.ds(s * RPT, RPT)])


@functools.partial(
    pl.kernel,
    out_type=(
        jax.ShapeDtypeStruct((NP, F), jnp.float32),
        jax.ShapeDtypeStruct((NP, F), jnp.float32),
    ),
    mesh=_mesh,
    compiler_params=pltpu.CompilerParams(use_tc_tiling_on_sc=False),
    scratch_types=[
        pltpu.VMEM((NB, BLK), jnp.int32),
        pltpu.VMEM((NB, BLK), jnp.int32),
        pltpu.VMEM((BLK, F), jnp.float32),
        pltpu.VMEM((BLK, F), jnp.float32),
        pltpu.VMEM((RPT, F), jnp.float32),
        pltpu.VMEM_SHARED((NP, F), jnp.float32),
        pltpu.VMEM_SHARED((NP, F), jnp.float32),
        pltpu.SemaphoreType.DMA,
        pltpu.SemaphoreType.DMA,
    ],
)
def _sc_aggregate(u_hbm, src_hbm, dst_hbm, out0_hbm, out1_hbm,
                  srcv, dstv, rows0, rows1, zbuf, u_sh, aggsh, gsem0, gsem1):
    c = lax.axis_index("c")
    s = lax.axis_index("s")
    wid = c * 16 + s
    pltpu.sync_copy(src_hbm.at[pl.ds(wid * NB, NB)], srcv)
    pltpu.sync_copy(dst_hbm.at[pl.ds(wid * NB, NB)], dstv)
    pltpu.sync_copy(u_hbm.at[pl.ds(s * RPT, RPT)], u_sh.at[pl.ds(s * RPT, RPT)])

    def initz(i, carry):
        zbuf[i] = jnp.zeros((16,), jnp.float32)
        return carry

    lax.fori_loop(0, RPT, initz, 0)
    pltpu.sync_copy(zbuf, aggsh.at[pl.ds(s * RPT, RPT)])
    plsc.subcore_barrier()

    pltpu.async_copy(u_sh.at[srcv.at[0]], rows0, gsem0)

    def body(t, carry):
        j = t * 2
        pltpu.async_copy(u_sh.at[srcv.at[j + 1]], rows1, gsem1)
        pltpu.make_async_copy(u_sh.at[srcv.at[j]], rows0, gsem0).wait()
        pltpu.sync_copy(rows0, aggsh.at[dstv.at[j]], add=True)
        pltpu.async_copy(u_sh.at[srcv.at[j + 2]], rows0, gsem0)
        pltpu.make_async_copy(u_sh.at[srcv.at[j + 1]], rows1, gsem1).wait()
        pltpu.sync_copy(rows1, aggsh.at[dstv.at[j + 1]], add=True)
        return carry

    lax.fori_loop(0, NB // 2 - 1, body, 0)
    pltpu.async_copy(u_sh.at[srcv.at[NB - 1]], rows1, gsem1)
    pltpu.make_async_copy(u_sh.at[srcv.at[NB - 2]], rows0, gsem0).wait()
    pltpu.sync_copy(rows0, aggsh.at[dstv.at[NB - 2]], add=True)
    pltpu.make_async_copy(u_sh.at[srcv.at[NB - 1]], rows1, gsem1).wait()
    pltpu.sync_copy(rows1, aggsh.at[dstv.at[NB - 1]], add=True)

    plsc.subcore_barrier()

    @pl.when(c == 0)
    def _():
        pltpu.sync_copy(aggsh.at[pl.ds(s * RPT, RPT)],
                        out0_hbm.at[pl.ds(s * RPT, RPT)])

    @pl.when(c == 1)
    def _():
        pltpu.sync_copy(aggsh.at[pl.ds(s * RPT, RPT)],
                        out1_hbm.at[pl.ds(s * RPT, RPT)])


def _tc1_body(d0_ref, d1_ref, x_ref, w1_ref, u1_ref, dinv_ref):
    deg = d0_ref[...] + d1_ref[...] + 1.0
    dinv = lax.rsqrt(deg)
    dinv_ref[...] = dinv
    h = jnp.dot(x_ref[...], w1_ref[...], preferred_element_type=jnp.float32)
    u1_ref[...] = dinv * h


def _tc2_body(a0_ref, a1_ref, u1_ref, dinv_ref, b1_ref, u2_ref):
    dinv = dinv_ref[...]
    a = a0_ref[...] + a1_ref[...] + u1_ref[...]
    z = jnp.maximum(dinv * a + b1_ref[...], 0.0)
    u2_ref[...] = dinv * z


def _tc3_body(a0_ref, a1_ref, u2_ref, dinv_ref, w2_ref, b2_ref, g_ref,
              out_ref):
    pre = dinv_ref[...] * (a0_ref[...] + a1_ref[...] + u2_ref[...])
    logits = jnp.dot(pre, w2_ref[...], preferred_element_type=jnp.float32)
    logits = logits + b2_ref[...]
    e = jnp.exp(logits)
    s = jnp.dot(e, g_ref[...], preferred_element_type=jnp.float32)
    out_ref[...] = logits - jnp.log(s)


def kernel(x, edge_index, W1, b1, W2, b2):
    ei = edge_index.astype(jnp.int32)
    f32 = jnp.float32
    eye8 = jnp.eye(8, dtype=f32)

    src_flat, dst_flat = pl.pallas_call(
        _tc0_body,
        out_shape=(
            jax.ShapeDtypeStruct((E_PAD,), jnp.int32),
            jax.ShapeDtypeStruct((E_PAD,), jnp.int32),
        ),
    )(ei)
    src2d = src_flat.reshape(E_PAD // BLK, BLK)
    dst2d = dst_flat.reshape(E_PAD // BLK, BLK)

    deg0, deg1 = _sc_degree(dst2d)

    x128 = jnp.pad(x, ((0, NP - N), (0, 0))).reshape(NR, 1024)
    W1big = jnp.kron(eye8, W1)
    u1, dinv = pl.pallas_call(
        _tc1_body,
        out_shape=(
            jax.ShapeDtypeStruct((NR, 128), f32),
            jax.ShapeDtypeStruct((NR, 128), f32),
        ),
    )(deg0.reshape(NR, 128), deg1.reshape(NR, 128), x128, W1big)

    a10, a11 = _sc_aggregate(u1.reshape(NP, F), src2d, dst2d)

    b1big = jnp.tile(b1, 8).reshape(1, 128)
    u2 = pl.pallas_call(
        _tc2_body,
        out_shape=jax.ShapeDtypeStruct((NR, 128), f32),
    )(a10.reshape(NR, 128), a11.reshape(NR, 128), u1, dinv, b1big)

    a20, a21 = _sc_aggregate(u2.reshape(NP, F), src2d, dst2d)

    W2big = jnp.kron(eye8, W2)
    b2big = jnp.tile(b2, 8).reshape(1, 512)
    G = jnp.kron(eye8, jnp.ones((64, 64), f32))
    out512 = pl.pallas_call(
        _tc3_body,
        out_shape=jax.ShapeDtypeStruct((NR, 512), f32),
    )(a20.reshape(NR, 128), a21.reshape(NR, 128), u2, dinv, W2big, b2big, G)

    return out512.reshape(NP, 64)[:N]
